# baseline (speedup 1.0000x reference)
import jax
import jax.numpy as jnp
from jax.experimental import pallas as pl
from jax.experimental.pallas import tpu as pltpu

_B, _T, _D = 4, 4096, 1024
_A0, _A1 = 64, 64


def _body(w0_ref, w1_ref, out_ref):
    row = w0_ref[0, 0, 0, :]
    tile = w1_ref[0, 0, :, :]
    out_ref[...] = jnp.broadcast_to((row[None, :] + tile)[None, :, :],
                                    out_ref.shape)


def kernel(x, w0, w1):
    del x
    out = pl.pallas_call(
        _body,
        grid=(_A0,),
        in_specs=[
            pl.BlockSpec((1, 1, 1, _D), lambda i: (0, i, 0, 0)),
            pl.BlockSpec((1, 1, _A1, _D), lambda i: (0, 0, 0, 0)),
        ],
        out_specs=pl.BlockSpec((_B, _A1, _D), lambda i: (0, i, 0)),
        out_shape=jax.ShapeDtypeStruct((_B, _T, _D), jnp.float32),
    )(w0, w1)
    return out


# speedup vs baseline: 1.4710x; 1.4710x over previous
import jax
import jax.numpy as jnp
from jax.experimental import pallas as pl
from jax.experimental.pallas import tpu as pltpu

_B, _T, _D = 4, 4096, 1024
_A0, _A1 = 64, 64


_RPB = 8
_TB = _RPB * _A1


def _body(w0_ref, w1_ref, out_ref):
    rows = w0_ref[0, :, 0, :]
    tile = w1_ref[0, 0, :, :]
    s = rows[:, None, :] + tile[None, :, :]
    out_ref[0] = s.reshape(_TB, _D)


def kernel(x, w0, w1):
    del x
    out = pl.pallas_call(
        _body,
        grid=(_B, _A0 // _RPB),
        in_specs=[
            pl.BlockSpec((1, _RPB, 1, _D), lambda b, k: (0, k, 0, 0)),
            pl.BlockSpec((1, 1, _A1, _D), lambda b, k: (0, 0, 0, 0)),
        ],
        out_specs=pl.BlockSpec((1, _TB, _D), lambda b, k: (b, k, 0)),
        out_shape=jax.ShapeDtypeStruct((_B, _T, _D), jnp.float32),
    )(w0, w1)
    return out


# speedup vs baseline: 1.8852x; 1.2815x over previous
import jax
import jax.numpy as jnp
from jax.experimental import pallas as pl
from jax.experimental.pallas import tpu as pltpu

_B, _T, _D = 4, 4096, 1024
_A0, _A1 = 64, 64


_RPB = 16
_TB = _RPB * _A1


def _body(w0_ref, w1_ref, out_ref):
    rows = w0_ref[0, :, 0, :]
    tile = w1_ref[0, 0, :, :]
    s = rows[:, None, :] + tile[None, :, :]
    out_ref[0] = s.reshape(_TB, _D)


def kernel(x, w0, w1):
    del x
    out = pl.pallas_call(
        _body,
        grid=(_B, _A0 // _RPB),
        in_specs=[
            pl.BlockSpec((1, _RPB, 1, _D), lambda b, k: (0, k, 0, 0)),
            pl.BlockSpec((1, 1, _A1, _D), lambda b, k: (0, 0, 0, 0)),
        ],
        out_specs=pl.BlockSpec((1, _TB, _D), lambda b, k: (b, k, 0)),
        out_shape=jax.ShapeDtypeStruct((_B, _T, _D), jnp.float32),
    )(w0, w1)
    return out
